# baseline (device time: 359003 ns/iter reference)
import os

import jax
import jax.numpy as jnp
from jax import lax
from jax.experimental import pallas as pl
from jax.experimental.pallas import tpu as pltpu

_SKIP_A2A = os.environ.get("SKIP_A2A") == "1"
_SKIP_ATTN = os.environ.get("SKIP_ATTN") == "1"
_SKIP_AR = os.environ.get("SKIP_AR") == "1"
_FP8 = os.environ.get("FP8") == "1"
_WIN = int(os.environ.get("A2A_WINDOW", "0"))


def kernel(x, Wq, K_ext, V_ext, Wo):
    B, Sq, Din = x.shape
    _, Skv_l, Hq, Dh = K_ext.shape
    Hl = Wq.shape[1] // Dh
    W = Hq // Hl
    Dout = Wo.shape[1]
    QB = 64
    NQB = Sq // QB

    bf16 = jnp.bfloat16
    wire_t = jnp.float8_e4m3fn if _FP8 else bf16
    xb = x.astype(bf16)
    Wqb = Wq.astype(bf16)
    Wob = Wo.astype(bf16)
    Kp = jnp.transpose(K_ext.astype(wire_t), (2, 0, 1, 3)).reshape(
        W, Hl, B, Skv_l, Dh
    )
    Vp = jnp.transpose(V_ext.astype(wire_t), (2, 0, 1, 3)).reshape(
        W, Hl, B, Skv_l, Dh
    )
    KVp = jnp.stack([Kp, Vp], axis=1).reshape(
        4, 4, 2, Hl, B, Skv_l, Dh
    )

    R = B * Sq
    SH = R // W

    def body(
        x_ref, wq_ref, kvp_ref, wo_ref, out_ref,
        stage_a, kv_buf, acc_ref, stage_ref, rs_buf, ag_ref, q_ref, ctx_ref,
        kv_send_sems, kv_recv_sems, p_send_sems, p_recv_sems,
    ):
        me = lax.axis_index("i")
        pm = me % 4
        zm = me // 4

        a_sends = []
        for dz in range(1, 4):
            if _SKIP_A2A:
                break
            dst_z = (zm + dz) % 4
            rdma = pltpu.make_async_remote_copy(
                src_ref=kvp_ref.at[pl.ds(dst_z, 1)],
                dst_ref=stage_a.at[pl.ds(dz, 1)],
                send_sem=kv_send_sems.at[dz],
                recv_sem=kv_recv_sems.at[dz],
                device_id=(dst_z * 4 + pm,),
                device_id_type=pl.DeviceIdType.MESH,
            )
            rdma.start()
            a_sends.append(rdma)

        stage_a[pl.ds(0, 1)] = kvp_ref[pl.ds(zm, 1)]

        for b in range(B):
            q_ref[b] = jnp.dot(
                x_ref[b], wq_ref[...], preferred_element_type=jnp.float32
            ).astype(bf16)

        for r in a_sends:
            r.wait_recv()

        b_sends = []
        for dp in range(1, 4):
            if _SKIP_A2A:
                break
            dst_p = (pm + dp) % 4
            rdma = pltpu.make_async_remote_copy(
                src_ref=stage_a.at[:, pl.ds(dst_p, 1)],
                dst_ref=kv_buf.at[:, pl.ds(dp, 1)],
                send_sem=kv_send_sems.at[4 + dp],
                recv_sem=kv_recv_sems.at[4 + dp],
                device_id=(zm * 4 + dst_p,),
                device_id_type=pl.DeviceIdType.MESH,
            )
            rdma.start()
            b_sends.append(rdma)

        kv_buf[:, pl.ds(0, 1)] = stage_a[:, pl.ds(pm, 1)]

        for r in b_sends:
            r.wait_recv()
        for r in a_sends:
            r.wait_send()
        for r in b_sends:
            r.wait_send()

        if _SKIP_ATTN:
            ctx_ref[...] = q_ref[...]
        for b in range(B if not _SKIP_ATTN else 0):
            for h in range(Hl):
                for qb in range(NQB):
                    q = q_ref[b, qb * QB:(qb + 1) * QB, h * Dh:(h + 1) * Dh]
                    k = kv_buf[:, :, 0, h, b, qb * QB:(qb + 1) * QB, :].reshape(
                        W * QB, Dh
                    ).astype(bf16)
                    v = kv_buf[:, :, 1, h, b, qb * QB:(qb + 1) * QB, :].reshape(
                        W * QB, Dh
                    ).astype(bf16)
                    scores = lax.dot_general(
                        q, k, (((1,), (1,)), ((), ())),
                        preferred_element_type=jnp.float32,
                    ) * 0.125
                    m = jnp.max(scores, axis=-1, keepdims=True)
                    w = jnp.exp(scores - m)
                    w = (w / jnp.sum(w, axis=-1, keepdims=True)).astype(bf16)
                    ctx = lax.dot_general(
                        w, v, (((1,), (0,)), ((), ())),
                        preferred_element_type=jnp.float32,
                    ).astype(bf16)
                    ctx_ref[b, qb * QB:(qb + 1) * QB, h * Dh:(h + 1) * Dh] = ctx

        for b in range(B):
            acc_ref[pl.ds(b * Sq, Sq)] = jnp.dot(
                ctx_ref[b], wo_ref[...], preferred_element_type=jnp.float32
            )

        if _SKIP_AR:
            out_ref[...] = acc_ref[...].reshape(B, Sq, Dout)
            return

        n_rounds = W.bit_length() - 1
        rs_offs = []
        off = 0
        for r in range(n_rounds):
            rs_offs.append(off)
            off += R >> (r + 1)

        lo = me * 0
        prev = None
        for r in range(n_rounds):
            m = W >> (r + 1)
            s2 = R >> (r + 1)
            bit = (me >> (n_rounds - 1 - r)) & 1
            keep_lo = lo + bit * s2
            sent_lo = lo + (1 - bit) * s2
            partner = me ^ m
            stage_ref[pl.ds(0, s2)] = acc_ref[pl.ds(sent_lo, s2)].astype(bf16)
            rdma = pltpu.make_async_remote_copy(
                src_ref=stage_ref.at[pl.ds(0, s2)],
                dst_ref=rs_buf.at[pl.ds(rs_offs[r], s2)],
                send_sem=p_send_sems.at[r],
                recv_sem=p_recv_sems.at[r],
                device_id=(partner,),
                device_id_type=pl.DeviceIdType.MESH,
            )
            rdma.start()
            rdma.wait_recv()
            acc_ref[pl.ds(keep_lo, s2)] = acc_ref[pl.ds(keep_lo, s2)] + rs_buf[
                pl.ds(rs_offs[r], s2)
            ].astype(jnp.float32)
            rdma.wait_send()
            lo = keep_lo

        ag_ref[pl.ds(SH * me, SH)] = acc_ref[pl.ds(lo, SH)].astype(bf16)
        ag_rdmas = []
        for r in range(n_rounds):
            m = 1 << r
            span = SH * m
            kmask = (W - 1) ^ (m - 1)
            vlo = SH * (me & kmask)
            partner = me ^ m
            rdma = pltpu.make_async_remote_copy(
                src_ref=ag_ref.at[pl.ds(vlo, span)],
                dst_ref=ag_ref.at[pl.ds(vlo, span)],
                send_sem=p_send_sems.at[4 + r],
                recv_sem=p_recv_sems.at[4 + r],
                device_id=(partner,),
                device_id_type=pl.DeviceIdType.MESH,
            )
            rdma.start()
            rdma.wait_recv()
            ag_rdmas.append(rdma)
        for rdma in ag_rdmas:
            rdma.wait_send()

        out_ref[...] = ag_ref[...].astype(jnp.float32).reshape(B, Sq, Dout)

    return pl.pallas_call(
        body,
        out_shape=jax.ShapeDtypeStruct((B, Sq, Dout), jnp.float32),
        in_specs=[pl.BlockSpec(memory_space=pltpu.VMEM)] * 4,
        out_specs=pl.BlockSpec(memory_space=pltpu.VMEM),
        scratch_shapes=[
            pltpu.VMEM((4, 4, 2, Hl, B, Skv_l, Dh), wire_t),
            pltpu.VMEM((4, 4, 2, Hl, B, Skv_l, Dh), wire_t),
            pltpu.VMEM((R, Dout), jnp.float32),
            pltpu.VMEM((R // 2, Dout), bf16),
            pltpu.VMEM((R - SH, Dout), bf16),
            pltpu.VMEM((R, Dout), bf16),
            pltpu.VMEM((B, Sq, Hl * Dh), bf16),
            pltpu.VMEM((B, Sq, Hl * Dh), bf16),
            pltpu.SemaphoreType.DMA((W,)),
            pltpu.SemaphoreType.DMA((W,)),
            pltpu.SemaphoreType.DMA((8,)),
            pltpu.SemaphoreType.DMA((8,)),
        ],
        compiler_params=pltpu.CompilerParams(
            vmem_limit_bytes=100 * 1024 * 1024,
        ),
    )(xb, Wqb, KVp, Wob)


# device time: 288374 ns/iter; 1.2449x vs baseline; 1.2449x over previous
import os

import jax
import jax.numpy as jnp
from jax import lax
from jax.experimental import pallas as pl
from jax.experimental.pallas import tpu as pltpu

_SKIP_A2A = os.environ.get("SKIP_A2A") == "1"
_SKIP_ATTN = os.environ.get("SKIP_ATTN") == "1"
_SKIP_AR = os.environ.get("SKIP_AR") == "1"
_FP8 = os.environ.get("FP8") == "1"
_WIN = int(os.environ.get("A2A_WINDOW", "0"))


def kernel(x, Wq, K_ext, V_ext, Wo):
    B, Sq, Din = x.shape
    _, Skv_l, Hq, Dh = K_ext.shape
    Hl = Wq.shape[1] // Dh
    W = Hq // Hl
    Dout = Wo.shape[1]
    QB = 64
    NQB = Sq // QB

    bf16 = jnp.bfloat16
    wire_t = jnp.float8_e4m3fn if _FP8 else bf16
    xb = x.astype(bf16)
    Wqb = Wq.astype(bf16)
    Wob = Wo.astype(bf16)
    Kp = jnp.transpose(K_ext.astype(wire_t), (2, 0, 1, 3)).reshape(
        W, Hl, B, Skv_l, Dh
    )
    Vp = jnp.transpose(V_ext.astype(wire_t), (2, 0, 1, 3)).reshape(
        W, Hl, B, Skv_l, Dh
    )
    KVp = jnp.stack([Kp, Vp], axis=1)

    R = B * Sq
    SH = R // W

    def body(
        x_ref, wq_ref, kvp_ref, wo_ref, out_ref,
        kv_buf, acc_ref, stage_ref, rs_buf, ag_ref, q_ref, ctx_ref,
        kv_send_sems, kv_recv_sems, p_send_sems, p_recv_sems,
    ):
        me = lax.axis_index("i")

        kv_sends = []
        for d in range(1, W):
            if _SKIP_A2A:
                break
            dst = (me + d) % W
            rdma = pltpu.make_async_remote_copy(
                src_ref=kvp_ref.at[pl.ds(dst, 1)],
                dst_ref=kv_buf.at[pl.ds(d, 1)],
                send_sem=kv_send_sems.at[d],
                recv_sem=kv_recv_sems.at[d],
                device_id=(dst,),
                device_id_type=pl.DeviceIdType.MESH,
            )
            rdma.start()
            kv_sends.append(rdma)

        kv_buf[pl.ds(0, 1)] = kvp_ref[pl.ds(me, 1)]

        for b in range(B):
            q_ref[b] = jnp.dot(
                x_ref[b], wq_ref[...], preferred_element_type=jnp.float32
            ).astype(bf16)

        HW = W // 2

        def _flash_part(lo_slot, b, h, qb):
            q = q_ref[b, qb * QB:(qb + 1) * QB, h * Dh:(h + 1) * Dh]
            k = kv_buf[
                lo_slot:lo_slot + HW, 0, h, b, qb * QB:(qb + 1) * QB, :
            ].reshape(HW * QB, Dh).astype(bf16)
            v = kv_buf[
                lo_slot:lo_slot + HW, 1, h, b, qb * QB:(qb + 1) * QB, :
            ].reshape(HW * QB, Dh).astype(bf16)
            s = lax.dot_general(
                q, k, (((1,), (1,)), ((), ())),
                preferred_element_type=jnp.float32,
            ) * 0.125
            m = jnp.max(s, axis=-1, keepdims=True)
            e = jnp.exp(s - m)
            l = jnp.sum(e, axis=-1, keepdims=True)
            c = lax.dot_general(
                e.astype(bf16), v, (((1,), (0,)), ((), ())),
                preferred_element_type=jnp.float32,
            )
            return m, l, c

        if _SKIP_ATTN:
            ctx_ref[...] = q_ref[...]

        for r in kv_sends[:HW - 1]:
            r.wait_recv()
        part1 = {}
        for b in range(B if not _SKIP_ATTN else 0):
            for h in range(Hl):
                for qb in range(NQB):
                    part1[(b, h, qb)] = _flash_part(0, b, h, qb)

        for r in kv_sends[HW - 1:]:
            r.wait_recv()
        for r in kv_sends:
            r.wait_send()
        for b in range(B if not _SKIP_ATTN else 0):
            for h in range(Hl):
                for qb in range(NQB):
                    m1, l1, c1 = part1[(b, h, qb)]
                    m2, l2, c2 = _flash_part(HW, b, h, qb)
                    m = jnp.maximum(m1, m2)
                    a1 = jnp.exp(m1 - m)
                    a2 = jnp.exp(m2 - m)
                    ctx = (c1 * a1 + c2 * a2) / (l1 * a1 + l2 * a2)
                    ctx_ref[
                        b, qb * QB:(qb + 1) * QB, h * Dh:(h + 1) * Dh
                    ] = ctx.astype(bf16)

        for b in range(B):
            acc_ref[pl.ds(b * Sq, Sq)] = jnp.dot(
                ctx_ref[b], wo_ref[...], preferred_element_type=jnp.float32
            )

        if _SKIP_AR:
            out_ref[...] = acc_ref[...].reshape(B, Sq, Dout)
            return

        n_rounds = W.bit_length() - 1
        rs_offs = []
        off = 0
        for r in range(n_rounds):
            rs_offs.append(off)
            off += R >> (r + 1)

        lo = me * 0
        prev = None
        for r in range(n_rounds):
            m = W >> (r + 1)
            s2 = R >> (r + 1)
            bit = (me >> (n_rounds - 1 - r)) & 1
            keep_lo = lo + bit * s2
            sent_lo = lo + (1 - bit) * s2
            partner = me ^ m
            stage_ref[pl.ds(0, s2)] = acc_ref[pl.ds(sent_lo, s2)].astype(bf16)
            rdma = pltpu.make_async_remote_copy(
                src_ref=stage_ref.at[pl.ds(0, s2)],
                dst_ref=rs_buf.at[pl.ds(rs_offs[r], s2)],
                send_sem=p_send_sems.at[r],
                recv_sem=p_recv_sems.at[r],
                device_id=(partner,),
                device_id_type=pl.DeviceIdType.MESH,
            )
            rdma.start()
            rdma.wait_recv()
            acc_ref[pl.ds(keep_lo, s2)] = acc_ref[pl.ds(keep_lo, s2)] + rs_buf[
                pl.ds(rs_offs[r], s2)
            ].astype(jnp.float32)
            rdma.wait_send()
            lo = keep_lo

        ag_ref[pl.ds(SH * me, SH)] = acc_ref[pl.ds(lo, SH)].astype(bf16)
        ag_rdmas = []
        for r in range(n_rounds):
            m = 1 << r
            span = SH * m
            kmask = (W - 1) ^ (m - 1)
            vlo = SH * (me & kmask)
            partner = me ^ m
            rdma = pltpu.make_async_remote_copy(
                src_ref=ag_ref.at[pl.ds(vlo, span)],
                dst_ref=ag_ref.at[pl.ds(vlo, span)],
                send_sem=p_send_sems.at[4 + r],
                recv_sem=p_recv_sems.at[4 + r],
                device_id=(partner,),
                device_id_type=pl.DeviceIdType.MESH,
            )
            rdma.start()
            rdma.wait_recv()
            ag_rdmas.append(rdma)
        for rdma in ag_rdmas:
            rdma.wait_send()

        out_ref[...] = ag_ref[...].astype(jnp.float32).reshape(B, Sq, Dout)

    return pl.pallas_call(
        body,
        out_shape=jax.ShapeDtypeStruct((B, Sq, Dout), jnp.float32),
        in_specs=[pl.BlockSpec(memory_space=pltpu.VMEM)] * 4,
        out_specs=pl.BlockSpec(memory_space=pltpu.VMEM),
        scratch_shapes=[
            pltpu.VMEM((W, 2, Hl, B, Skv_l, Dh), wire_t),
            pltpu.VMEM((R, Dout), jnp.float32),
            pltpu.VMEM((R // 2, Dout), bf16),
            pltpu.VMEM((R - SH, Dout), bf16),
            pltpu.VMEM((R, Dout), bf16),
            pltpu.VMEM((B, Sq, Hl * Dh), bf16),
            pltpu.VMEM((B, Sq, Hl * Dh), bf16),
            pltpu.SemaphoreType.DMA((W,)),
            pltpu.SemaphoreType.DMA((W,)),
            pltpu.SemaphoreType.DMA((8,)),
            pltpu.SemaphoreType.DMA((8,)),
        ],
    )(xb, Wqb, KVp, Wob)


# device time: 238404 ns/iter; 1.5059x vs baseline; 1.2096x over previous
import os

import jax
import jax.numpy as jnp
from jax import lax
from jax.experimental import pallas as pl
from jax.experimental.pallas import tpu as pltpu

_SKIP_A2A = os.environ.get("SKIP_A2A") == "1"
_SKIP_ATTN = os.environ.get("SKIP_ATTN") == "1"
_SKIP_AR = os.environ.get("SKIP_AR") == "1"
_NO_VFP8 = os.environ.get("NO_VFP8") == "1"


def kernel(x, Wq, K_ext, V_ext, Wo):
    B, Sq, Din = x.shape
    _, Skv_l, Hq, Dh = K_ext.shape
    Hl = Wq.shape[1] // Dh
    W = Hq // Hl
    Dout = Wo.shape[1]
    QB = 64
    NQB = Sq // QB

    bf16 = jnp.bfloat16
    k_wire = bf16 if _NO_VFP8 else jnp.float8_e4m3fn
    v_wire = bf16
    xb = x.astype(bf16)
    Wqb = Wq.astype(bf16)
    Wob = Wo.astype(bf16)
    Kp = jnp.transpose(K_ext.astype(k_wire), (2, 0, 1, 3)).reshape(
        W, Hl, B, Skv_l, Dh
    )
    Vp = jnp.transpose(V_ext.astype(v_wire), (2, 0, 1, 3)).reshape(
        W, Hl, B, Skv_l, Dh
    )

    R = B * Sq
    SH = R // W

    def body(
        x_ref, wq_ref, kp_ref, vp_ref, wo_ref, out_ref,
        k_buf, v_buf, acc_ref, stage_ref, rs_buf, ag_ref, q_ref, ctx_ref,
        k_send_sems, k_recv_sems, v_send_sems, v_recv_sems,
        p_send_sems, p_recv_sems,
    ):
        me = lax.axis_index("i")

        kv_sends = []
        for d in range(1, W):
            if _SKIP_A2A:
                break
            dst = (me + d) % W
            for src_ref, dst_ref, ss, rs in (
                (kp_ref, k_buf, k_send_sems, k_recv_sems),
                (vp_ref, v_buf, v_send_sems, v_recv_sems),
            ):
                rdma = pltpu.make_async_remote_copy(
                    src_ref=src_ref.at[pl.ds(dst, 1)],
                    dst_ref=dst_ref.at[pl.ds(d, 1)],
                    send_sem=ss.at[d],
                    recv_sem=rs.at[d],
                    device_id=(dst,),
                    device_id_type=pl.DeviceIdType.MESH,
                )
                rdma.start()
                kv_sends.append(rdma)

        k_buf[pl.ds(0, 1)] = kp_ref[pl.ds(me, 1)]
        v_buf[pl.ds(0, 1)] = vp_ref[pl.ds(me, 1)]

        for b in range(B):
            q_ref[b] = jnp.dot(
                x_ref[b], wq_ref[...], preferred_element_type=jnp.float32
            ).astype(bf16)

        HW = W // 2

        def _flash_part(lo_slot, b, h, qb):
            q = q_ref[b, qb * QB:(qb + 1) * QB, h * Dh:(h + 1) * Dh]
            k = k_buf[
                lo_slot:lo_slot + HW, h, b, qb * QB:(qb + 1) * QB, :
            ].reshape(HW * QB, Dh).astype(bf16)
            v = v_buf[
                lo_slot:lo_slot + HW, h, b, qb * QB:(qb + 1) * QB, :
            ].reshape(HW * QB, Dh)
            s = lax.dot_general(
                q, k, (((1,), (1,)), ((), ())),
                preferred_element_type=jnp.float32,
            ) * 0.125
            m = jnp.max(s, axis=-1, keepdims=True)
            e = jnp.exp(s - m)
            l = jnp.sum(e, axis=-1, keepdims=True)
            c = lax.dot_general(
                e.astype(bf16), v, (((1,), (0,)), ((), ())),
                preferred_element_type=jnp.float32,
            )
            return m, l, c

        if _SKIP_ATTN:
            ctx_ref[...] = q_ref[...]

        for r in kv_sends[:2 * (HW - 1)]:
            r.wait_recv()
        part1 = {}
        for b in range(B if not _SKIP_ATTN else 0):
            for h in range(Hl):
                for qb in range(NQB):
                    part1[(b, h, qb)] = _flash_part(0, b, h, qb)

        for r in kv_sends[2 * (HW - 1):]:
            r.wait_recv()
        for r in kv_sends:
            r.wait_send()
        for b in range(B if not _SKIP_ATTN else 0):
            for h in range(Hl):
                for qb in range(NQB):
                    m1, l1, c1 = part1[(b, h, qb)]
                    m2, l2, c2 = _flash_part(HW, b, h, qb)
                    m = jnp.maximum(m1, m2)
                    a1 = jnp.exp(m1 - m)
                    a2 = jnp.exp(m2 - m)
                    ctx = (c1 * a1 + c2 * a2) / (l1 * a1 + l2 * a2)
                    ctx_ref[
                        b, qb * QB:(qb + 1) * QB, h * Dh:(h + 1) * Dh
                    ] = ctx.astype(bf16)

        for b in range(B):
            acc_ref[pl.ds(b * Sq, Sq)] = jnp.dot(
                ctx_ref[b], wo_ref[...], preferred_element_type=jnp.float32
            )

        if _SKIP_AR:
            out_ref[...] = acc_ref[...].reshape(B, Sq, Dout)
            return

        n_rounds = W.bit_length() - 1
        rs_offs = []
        off = 0
        for r in range(n_rounds):
            rs_offs.append(off)
            off += R >> (r + 1)

        lo = me * 0
        prev = None
        for r in range(n_rounds):
            m = W >> (r + 1)
            s2 = R >> (r + 1)
            bit = (me >> (n_rounds - 1 - r)) & 1
            keep_lo = lo + bit * s2
            sent_lo = lo + (1 - bit) * s2
            partner = me ^ m
            stage_ref[pl.ds(0, s2)] = acc_ref[pl.ds(sent_lo, s2)].astype(bf16)
            rdma = pltpu.make_async_remote_copy(
                src_ref=stage_ref.at[pl.ds(0, s2)],
                dst_ref=rs_buf.at[pl.ds(rs_offs[r], s2)],
                send_sem=p_send_sems.at[r],
                recv_sem=p_recv_sems.at[r],
                device_id=(partner,),
                device_id_type=pl.DeviceIdType.MESH,
            )
            rdma.start()
            rdma.wait_recv()
            acc_ref[pl.ds(keep_lo, s2)] = acc_ref[pl.ds(keep_lo, s2)] + rs_buf[
                pl.ds(rs_offs[r], s2)
            ].astype(jnp.float32)
            rdma.wait_send()
            lo = keep_lo

        ag_ref[pl.ds(SH * me, SH)] = acc_ref[pl.ds(lo, SH)].astype(bf16)
        ag_rdmas = []
        for r in range(n_rounds):
            m = 1 << r
            span = SH * m
            kmask = (W - 1) ^ (m - 1)
            vlo = SH * (me & kmask)
            partner = me ^ m
            rdma = pltpu.make_async_remote_copy(
                src_ref=ag_ref.at[pl.ds(vlo, span)],
                dst_ref=ag_ref.at[pl.ds(vlo, span)],
                send_sem=p_send_sems.at[4 + r],
                recv_sem=p_recv_sems.at[4 + r],
                device_id=(partner,),
                device_id_type=pl.DeviceIdType.MESH,
            )
            rdma.start()
            rdma.wait_recv()
            ag_rdmas.append(rdma)
        for rdma in ag_rdmas:
            rdma.wait_send()

        out_ref[...] = ag_ref[...].astype(jnp.float32).reshape(B, Sq, Dout)

    return pl.pallas_call(
        body,
        out_shape=jax.ShapeDtypeStruct((B, Sq, Dout), jnp.float32),
        in_specs=[pl.BlockSpec(memory_space=pltpu.VMEM)] * 5,
        out_specs=pl.BlockSpec(memory_space=pltpu.VMEM),
        scratch_shapes=[
            pltpu.VMEM((W, Hl, B, Skv_l, Dh), k_wire),
            pltpu.VMEM((W, Hl, B, Skv_l, Dh), v_wire),
            pltpu.VMEM((R, Dout), jnp.float32),
            pltpu.VMEM((R // 2, Dout), bf16),
            pltpu.VMEM((R - SH, Dout), bf16),
            pltpu.VMEM((R, Dout), bf16),
            pltpu.VMEM((B, Sq, Hl * Dh), bf16),
            pltpu.VMEM((B, Sq, Hl * Dh), bf16),
            pltpu.SemaphoreType.DMA((W,)),
            pltpu.SemaphoreType.DMA((W,)),
            pltpu.SemaphoreType.DMA((W,)),
            pltpu.SemaphoreType.DMA((W,)),
            pltpu.SemaphoreType.DMA((8,)),
            pltpu.SemaphoreType.DMA((8,)),
        ],
        compiler_params=pltpu.CompilerParams(
            vmem_limit_bytes=100 * 1024 * 1024,
        ),
    )(xb, Wqb, Kp, Vp, Wob)


# device time: 183240 ns/iter; 1.9592x vs baseline; 1.3010x over previous
import os

import jax
import jax.numpy as jnp
from jax import lax
from jax.experimental import pallas as pl
from jax.experimental.pallas import tpu as pltpu

_SKIP_A2A = os.environ.get("SKIP_A2A") == "1"
_SKIP_ATTN = os.environ.get("SKIP_ATTN") == "1"
_SKIP_AR = os.environ.get("SKIP_AR") == "1"
_NO_VFP8 = os.environ.get("NO_VFP8") == "1"


def kernel(x, Wq, K_ext, V_ext, Wo):
    B, Sq, Din = x.shape
    _, Skv_l, Hq, Dh = K_ext.shape
    Hl = Wq.shape[1] // Dh
    W = Hq // Hl
    Dout = Wo.shape[1]
    QB = 64
    NQB = Sq // QB

    bf16 = jnp.bfloat16
    quant = not _NO_VFP8
    Q_SCALE = 32.0
    wire = jnp.int8 if quant else bf16

    def _pack(t):
        tq = (
            jnp.clip(jnp.round(t * Q_SCALE), -127, 127).astype(jnp.int8)
            if quant
            else t.astype(bf16)
        )
        return jnp.transpose(tq, (2, 0, 1, 3)).reshape(W, Hl, B, Skv_l, Dh)

    xb = x.astype(bf16)
    Wqb = Wq.astype(bf16)
    Wob = Wo.astype(bf16)
    Kp = _pack(K_ext)
    Vp = _pack(V_ext)

    R = B * Sq
    SH = R // W

    def body(
        x_ref, wq_ref, kp_ref, vp_ref, wo_ref, out_ref,
        k_buf, v_buf, acc_ref, stage_ref, rs_buf, ag_ref, q_ref, ctx_ref,
        k_send_sems, k_recv_sems, v_send_sems, v_recv_sems,
        p_send_sems, p_recv_sems,
    ):
        me = lax.axis_index("i")

        kv_sends = []
        for d in range(1, W):
            if _SKIP_A2A:
                break
            dst = (me + d) % W
            for src_ref, dst_ref, ss, rs in (
                (kp_ref, k_buf, k_send_sems, k_recv_sems),
                (vp_ref, v_buf, v_send_sems, v_recv_sems),
            ):
                rdma = pltpu.make_async_remote_copy(
                    src_ref=src_ref.at[pl.ds(dst, 1)],
                    dst_ref=dst_ref.at[pl.ds(d, 1)],
                    send_sem=ss.at[d],
                    recv_sem=rs.at[d],
                    device_id=(dst,),
                    device_id_type=pl.DeviceIdType.MESH,
                )
                rdma.start()
                kv_sends.append(rdma)

        k_buf[pl.ds(0, 1)] = kp_ref[pl.ds(me, 1)]
        v_buf[pl.ds(0, 1)] = vp_ref[pl.ds(me, 1)]

        for b in range(B):
            q_ref[b] = jnp.dot(
                x_ref[b], wq_ref[...], preferred_element_type=jnp.float32
            ).astype(bf16)

        HW = W // 2

        def _flash_part(lo_slot, b, h, qb):
            q = q_ref[b, qb * QB:(qb + 1) * QB, h * Dh:(h + 1) * Dh]
            k = k_buf[
                lo_slot:lo_slot + HW, h, b, qb * QB:(qb + 1) * QB, :
            ].reshape(HW * QB, Dh).astype(bf16)
            v = v_buf[
                lo_slot:lo_slot + HW, h, b, qb * QB:(qb + 1) * QB, :
            ].reshape(HW * QB, Dh).astype(bf16)
            s = lax.dot_general(
                q, k, (((1,), (1,)), ((), ())),
                preferred_element_type=jnp.float32,
            ) * (0.125 / Q_SCALE if quant else 0.125)
            m = jnp.max(s, axis=-1, keepdims=True)
            e = jnp.exp(s - m)
            l = jnp.sum(e, axis=-1, keepdims=True)
            c = lax.dot_general(
                e.astype(bf16), v, (((1,), (0,)), ((), ())),
                preferred_element_type=jnp.float32,
            )
            return m, l, c

        if _SKIP_ATTN:
            ctx_ref[...] = q_ref[...]

        for r in kv_sends[:2 * (HW - 1)]:
            r.wait_recv()
        part1 = {}
        for b in range(B if not _SKIP_ATTN else 0):
            for h in range(Hl):
                for qb in range(NQB):
                    part1[(b, h, qb)] = _flash_part(0, b, h, qb)

        for r in kv_sends[2 * (HW - 1):]:
            r.wait_recv()
        for r in kv_sends:
            r.wait_send()
        for b in range(B if not _SKIP_ATTN else 0):
            for h in range(Hl):
                for qb in range(NQB):
                    m1, l1, c1 = part1[(b, h, qb)]
                    m2, l2, c2 = _flash_part(HW, b, h, qb)
                    m = jnp.maximum(m1, m2)
                    a1 = jnp.exp(m1 - m)
                    a2 = jnp.exp(m2 - m)
                    ctx = (c1 * a1 + c2 * a2) / (l1 * a1 + l2 * a2)
                    if quant:
                        ctx = ctx * (1.0 / Q_SCALE)
                    ctx_ref[
                        b, qb * QB:(qb + 1) * QB, h * Dh:(h + 1) * Dh
                    ] = ctx.astype(bf16)

        for b in range(B):
            acc_ref[pl.ds(b * Sq, Sq)] = jnp.dot(
                ctx_ref[b], wo_ref[...], preferred_element_type=jnp.float32
            )

        if _SKIP_AR:
            out_ref[...] = acc_ref[...].reshape(B, Sq, Dout)
            return

        n_rounds = W.bit_length() - 1
        rs_offs = []
        off = 0
        for r in range(n_rounds):
            rs_offs.append(off)
            off += R >> (r + 1)

        lo = me * 0
        prev = None
        for r in range(n_rounds):
            m = W >> (r + 1)
            s2 = R >> (r + 1)
            bit = (me >> (n_rounds - 1 - r)) & 1
            keep_lo = lo + bit * s2
            sent_lo = lo + (1 - bit) * s2
            partner = me ^ m
            stage_ref[pl.ds(0, s2)] = acc_ref[pl.ds(sent_lo, s2)].astype(bf16)
            rdma = pltpu.make_async_remote_copy(
                src_ref=stage_ref.at[pl.ds(0, s2)],
                dst_ref=rs_buf.at[pl.ds(rs_offs[r], s2)],
                send_sem=p_send_sems.at[r],
                recv_sem=p_recv_sems.at[r],
                device_id=(partner,),
                device_id_type=pl.DeviceIdType.MESH,
            )
            rdma.start()
            rdma.wait_recv()
            acc_ref[pl.ds(keep_lo, s2)] = acc_ref[pl.ds(keep_lo, s2)] + rs_buf[
                pl.ds(rs_offs[r], s2)
            ].astype(jnp.float32)
            rdma.wait_send()
            lo = keep_lo

        ag_ref[pl.ds(SH * me, SH)] = acc_ref[pl.ds(lo, SH)].astype(bf16)
        ag_rdmas = []
        for r in range(n_rounds):
            m = 1 << r
            span = SH * m
            kmask = (W - 1) ^ (m - 1)
            vlo = SH * (me & kmask)
            partner = me ^ m
            rdma = pltpu.make_async_remote_copy(
                src_ref=ag_ref.at[pl.ds(vlo, span)],
                dst_ref=ag_ref.at[pl.ds(vlo, span)],
                send_sem=p_send_sems.at[4 + r],
                recv_sem=p_recv_sems.at[4 + r],
                device_id=(partner,),
                device_id_type=pl.DeviceIdType.MESH,
            )
            rdma.start()
            rdma.wait_recv()
            ag_rdmas.append(rdma)
        for rdma in ag_rdmas:
            rdma.wait_send()

        out_ref[...] = ag_ref[...].astype(jnp.float32).reshape(B, Sq, Dout)

    return pl.pallas_call(
        body,
        out_shape=jax.ShapeDtypeStruct((B, Sq, Dout), jnp.float32),
        in_specs=[pl.BlockSpec(memory_space=pltpu.VMEM)] * 5,
        out_specs=pl.BlockSpec(memory_space=pltpu.VMEM),
        scratch_shapes=[
            pltpu.VMEM((W, Hl, B, Skv_l, Dh), wire),
            pltpu.VMEM((W, Hl, B, Skv_l, Dh), wire),
            pltpu.VMEM((R, Dout), jnp.float32),
            pltpu.VMEM((R // 2, Dout), bf16),
            pltpu.VMEM((R - SH, Dout), bf16),
            pltpu.VMEM((R, Dout), bf16),
            pltpu.VMEM((B, Sq, Hl * Dh), bf16),
            pltpu.VMEM((B, Sq, Hl * Dh), bf16),
            pltpu.SemaphoreType.DMA((W,)),
            pltpu.SemaphoreType.DMA((W,)),
            pltpu.SemaphoreType.DMA((W,)),
            pltpu.SemaphoreType.DMA((W,)),
            pltpu.SemaphoreType.DMA((8,)),
            pltpu.SemaphoreType.DMA((8,)),
        ],
        compiler_params=pltpu.CompilerParams(
            vmem_limit_bytes=100 * 1024 * 1024,
        ),
    )(xb, Wqb, Kp, Vp, Wob)


# device time: 183072 ns/iter; 1.9610x vs baseline; 1.0009x over previous
import os

import jax
import jax.numpy as jnp
from jax import lax
from jax.experimental import pallas as pl
from jax.experimental.pallas import tpu as pltpu

_SKIP_A2A = os.environ.get("SKIP_A2A") == "1"
_SKIP_ATTN = os.environ.get("SKIP_ATTN") == "1"
_SKIP_AR = os.environ.get("SKIP_AR") == "1"
_NO_QUANT = os.environ.get("NO_QUANT") == "1"


def kernel(x, Wq, K_ext, V_ext, Wo):
    B, Sq, Din = x.shape
    _, Skv_l, Hq, Dh = K_ext.shape
    Hl = Wq.shape[1] // Dh
    W = Hq // Hl
    Dout = Wo.shape[1]
    QB = 64
    NQB = Sq // QB

    bf16 = jnp.bfloat16
    quant = not _NO_QUANT
    Q_SCALE = 32.0
    wire = jnp.int8 if quant else bf16

    def _pack(t):
        tq = (
            jnp.clip(jnp.round(t * Q_SCALE), -127, 127).astype(jnp.int8)
            if quant
            else t.astype(bf16)
        )
        return jnp.transpose(tq, (2, 0, 1, 3)).reshape(W, Hl, B, Skv_l, Dh)

    xb = x.astype(bf16)
    Wqb = Wq.astype(bf16)
    Wob = Wo.astype(bf16)
    Kp = _pack(K_ext)
    Vp = _pack(V_ext)

    R = B * Sq
    SH = R // W

    def body(
        x_ref, wq_ref, kp_ref, vp_ref, wo_ref, out_ref,
        k_buf, v_buf, acc_ref, stage_ref, rs_buf, ag_ref, q_ref, ctx_ref,
        k_send_sems, k_recv_sems, v_send_sems, v_recv_sems,
        p_send_sems, p_recv_sems,
    ):
        me = lax.axis_index("i")

        kv_sends = []
        for d in range(1, W):
            if _SKIP_A2A:
                break
            dst = (me + d) % W
            for src_ref, dst_ref, ss, rs in (
                (kp_ref, k_buf, k_send_sems, k_recv_sems),
                (vp_ref, v_buf, v_send_sems, v_recv_sems),
            ):
                rdma = pltpu.make_async_remote_copy(
                    src_ref=src_ref.at[pl.ds(dst, 1)],
                    dst_ref=dst_ref.at[pl.ds(d, 1)],
                    send_sem=ss.at[d],
                    recv_sem=rs.at[d],
                    device_id=(dst,),
                    device_id_type=pl.DeviceIdType.MESH,
                )
                rdma.start()
                kv_sends.append(rdma)

        k_buf[pl.ds(0, 1)] = kp_ref[pl.ds(me, 1)]
        v_buf[pl.ds(0, 1)] = vp_ref[pl.ds(me, 1)]

        for b in range(B):
            q_ref[b] = jnp.dot(
                x_ref[b], wq_ref[...], preferred_element_type=jnp.float32
            ).astype(bf16)

        HW = W // 2

        def _flash_part(lo_slot, b, h, qb):
            q = q_ref[b, qb * QB:(qb + 1) * QB, h * Dh:(h + 1) * Dh]
            k = k_buf[
                lo_slot:lo_slot + HW, h, b, qb * QB:(qb + 1) * QB, :
            ].reshape(HW * QB, Dh).astype(bf16)
            v = v_buf[
                lo_slot:lo_slot + HW, h, b, qb * QB:(qb + 1) * QB, :
            ].reshape(HW * QB, Dh).astype(bf16)
            s = lax.dot_general(
                q, k, (((1,), (1,)), ((), ())),
                preferred_element_type=jnp.float32,
            ) * (0.125 / Q_SCALE if quant else 0.125)
            m = jnp.max(s, axis=-1, keepdims=True)
            e = jnp.exp(s - m)
            l = jnp.sum(e, axis=-1, keepdims=True)
            c = lax.dot_general(
                e.astype(bf16), v, (((1,), (0,)), ((), ())),
                preferred_element_type=jnp.float32,
            )
            return m, l, c

        if _SKIP_ATTN:
            ctx_ref[...] = q_ref[...]

        for r in kv_sends[:2 * (HW - 1)]:
            r.wait_recv()
        part1 = {}
        for b in range(B if not _SKIP_ATTN else 0):
            for h in range(Hl):
                for qb in range(NQB):
                    part1[(b, h, qb)] = _flash_part(0, b, h, qb)

        for r in kv_sends[2 * (HW - 1):]:
            r.wait_recv()
        for r in kv_sends:
            r.wait_send()
        for b in range(B if not _SKIP_ATTN else 0):
            for h in range(Hl):
                for qb in range(NQB):
                    m1, l1, c1 = part1[(b, h, qb)]
                    m2, l2, c2 = _flash_part(HW, b, h, qb)
                    m = jnp.maximum(m1, m2)
                    a1 = jnp.exp(m1 - m)
                    a2 = jnp.exp(m2 - m)
                    ctx = (c1 * a1 + c2 * a2) / (l1 * a1 + l2 * a2)
                    if quant:
                        ctx = ctx * (1.0 / Q_SCALE)
                    ctx_ref[
                        b, qb * QB:(qb + 1) * QB, h * Dh:(h + 1) * Dh
                    ] = ctx.astype(bf16)

        for b in range(B):
            acc_ref[pl.ds(b * Sq, Sq)] = jnp.dot(
                ctx_ref[b], wo_ref[...], preferred_element_type=jnp.float32
            )

        if _SKIP_AR:
            out_ref[...] = acc_ref[...].reshape(B, Sq, Dout)
            return

        n_rounds = W.bit_length() - 1
        rs_offs = []
        off = 0
        for r in range(n_rounds):
            rs_offs.append(off)
            off += R >> (r + 1)

        lo = me * 0
        for r in range(n_rounds):
            m = W >> (r + 1)
            s2 = R >> (r + 1)
            bit = (me >> (n_rounds - 1 - r)) & 1
            keep_lo = lo + bit * s2
            sent_lo = lo + (1 - bit) * s2
            partner = me ^ m
            stage_ref[pl.ds(0, s2)] = acc_ref[pl.ds(sent_lo, s2)].astype(bf16)
            rdma = pltpu.make_async_remote_copy(
                src_ref=stage_ref.at[pl.ds(0, s2)],
                dst_ref=rs_buf.at[pl.ds(rs_offs[r], s2)],
                send_sem=p_send_sems.at[r],
                recv_sem=p_recv_sems.at[r],
                device_id=(partner,),
                device_id_type=pl.DeviceIdType.MESH,
            )
            rdma.start()
            rdma.wait_recv()
            acc_ref[pl.ds(keep_lo, s2)] = acc_ref[pl.ds(keep_lo, s2)] + rs_buf[
                pl.ds(rs_offs[r], s2)
            ].astype(jnp.float32)
            rdma.wait_send()
            lo = keep_lo

        ag_ref[pl.ds(SH * me, SH)] = acc_ref[pl.ds(lo, SH)].astype(bf16)
        ag_rdmas = []
        for r in range(n_rounds):
            m = 1 << r
            span = SH * m
            kmask = (W - 1) ^ (m - 1)
            vlo = SH * (me & kmask)
            partner = me ^ m
            rdma = pltpu.make_async_remote_copy(
                src_ref=ag_ref.at[pl.ds(vlo, span)],
                dst_ref=ag_ref.at[pl.ds(vlo, span)],
                send_sem=p_send_sems.at[4 + r],
                recv_sem=p_recv_sems.at[4 + r],
                device_id=(partner,),
                device_id_type=pl.DeviceIdType.MESH,
            )
            rdma.start()
            rdma.wait_recv()
            ag_rdmas.append(rdma)
        for rdma in ag_rdmas:
            rdma.wait_send()

        out_ref[...] = ag_ref[...].astype(jnp.float32).reshape(B, Sq, Dout)

    return pl.pallas_call(
        body,
        out_shape=jax.ShapeDtypeStruct((B, Sq, Dout), jnp.float32),
        in_specs=[pl.BlockSpec(memory_space=pltpu.VMEM)] * 5,
        out_specs=pl.BlockSpec(memory_space=pltpu.VMEM),
        scratch_shapes=[
            pltpu.VMEM((W, Hl, B, Skv_l, Dh), wire),
            pltpu.VMEM((W, Hl, B, Skv_l, Dh), wire),
            pltpu.VMEM((R, Dout), jnp.float32),
            pltpu.VMEM((R // 2, Dout), bf16),
            pltpu.VMEM((R - SH, Dout), bf16),
            pltpu.VMEM((R, Dout), bf16),
            pltpu.VMEM((B, Sq, Hl * Dh), bf16),
            pltpu.VMEM((B, Sq, Hl * Dh), bf16),
            pltpu.SemaphoreType.DMA((W,)),
            pltpu.SemaphoreType.DMA((W,)),
            pltpu.SemaphoreType.DMA((W,)),
            pltpu.SemaphoreType.DMA((W,)),
            pltpu.SemaphoreType.DMA((8,)),
            pltpu.SemaphoreType.DMA((8,)),
        ],
        compiler_params=pltpu.CompilerParams(
            vmem_limit_bytes=100 * 1024 * 1024,
        ),
    )(xb, Wqb, Kp, Vp, Wob)
